# baseline (speedup 1.0000x reference)
import functools

import jax
import jax.numpy as jnp
from jax import lax
from jax.experimental import pallas as pl
from jax.experimental.pallas import tpu as pltpu
from jax.experimental.pallas import tpu_sc as plsc

N = 10000
NP = 10240
E = 320000
EP = 327680
HID = 256
ED = 16
NUM_AT_PAD = 104


def _embed_body(an_ref, of_ref, emb_ref, x0_ref):
    an = an_ref[...]
    ids = jax.lax.broadcasted_iota(jnp.int32, (1, NUM_AT_PAD), 1)
    oh = (an == ids).astype(jnp.float32)
    emb = jnp.dot(oh, emb_ref[...], preferred_element_type=jnp.float32)
    B = emb.shape[0]
    x0_ref[...] = jnp.concatenate(
        [emb, of_ref[...], jnp.zeros((B, 56), jnp.float32)], axis=1)


def _embed(an2d, of_pad, emb_pad):
    B = 1024
    return pl.pallas_call(
        _embed_body,
        grid=(NP // B,),
        in_specs=[
            pl.BlockSpec((B, 1), lambda i: (i, 0)),
            pl.BlockSpec((B, 8), lambda i: (i, 0)),
            pl.BlockSpec((NUM_AT_PAD, 64), lambda i: (0, 0)),
        ],
        out_specs=pl.BlockSpec((B, 128), lambda i: (i, 0)),
        out_shape=jax.ShapeDtypeStruct((NP, 128), jnp.float32),
    )(an2d, of_pad, emb_pad)


def _edge_mlp_body(ea_ref, w_ref, *out_refs):
    e = jnp.dot(ea_ref[...], w_ref[...], preferred_element_type=jnp.float32)
    for k, r in enumerate(out_refs):
        r[...] = e[:, 128 * k:128 * (k + 1)]


def _edge_mlp(edge_attr, wcat):
    B = 2048
    return pl.pallas_call(
        _edge_mlp_body,
        grid=(EP // B,),
        in_specs=[
            pl.BlockSpec((B, ED), lambda i: (i, 0)),
            pl.BlockSpec((ED, 896), lambda i: (0, 0)),
        ],
        out_specs=[pl.BlockSpec((B, 128), lambda i: (i, 0)) for _ in range(7)],
        out_shape=[jax.ShapeDtypeStruct((EP, 128), jnp.float32)
                   for _ in range(7)],
    )(edge_attr, wcat)


_SC_RPT = NP // 16
_SC_WC = 80


def _sc_relu_add(xbuf, ebuf, C):
    @pl.loop(0, C, unroll=2)
    def _edge(i):
        for j in range(8):
            sl = pl.ds(j * 16, 16)
            xbuf[i, sl] = jnp.maximum(xbuf[i, sl] + ebuf[i, sl], 0.0)


def _sc_body(x0, x1, srcr, dstr, e0, e1, zrows, out0, out1,
             agg_sh, sidx, didx, xb0, xb1, eb0, eb1,
             sg0, sg1, se0, se1, ss0, ss1, spre,
             *, C, S, edge_split):
    c = lax.axis_index("c")
    s = lax.axis_index("s")
    xb = (xb0, xb1)
    eb = (eb0, eb1)
    sg = (sg0, sg1)
    se = (se0, se1)
    ss = (ss0, ss1)

    nworkers = 32 if edge_split else 16
    ept = EP // nworkers
    n = ept // C
    M = n // S
    rows_pt = ept // C
    wid = (c * 16 + s) if edge_split else s
    rbase = wid * rows_pt

    def idx_rows(m):
        return pl.ds(rbase + m * S, S)

    def issue_pre(m, slot):
        pltpu.async_copy(srcr.at[idx_rows(m)], sidx.at[slot], spre)
        pltpu.async_copy(dstr.at[idx_rows(m)], didx.at[slot], spre)

    def wait_pre(m, slot):
        pltpu.make_async_copy(srcr.at[idx_rows(m)], sidx.at[slot], spre).wait()
        pltpu.make_async_copy(dstr.at[idx_rows(m)], didx.at[slot], spre).wait()

    def issue_gather(slot, j, b, off):
        if edge_split:
            pltpu.async_copy(x0.at[sidx.at[slot, j]], xb[b], sg[b])
            pltpu.async_copy(e0.at[pl.ds(off, C)], eb[b], se[b])
        else:
            @pl.when(c == 0)
            def _():
                pltpu.async_copy(x0.at[sidx.at[slot, j]], xb[b], sg[b])
                pltpu.async_copy(e0.at[pl.ds(off, C)], eb[b], se[b])

            @pl.when(c == 1)
            def _():
                pltpu.async_copy(x1.at[sidx.at[slot, j]], xb[b], sg[b])
                pltpu.async_copy(e1.at[pl.ds(off, C)], eb[b], se[b])

    def wait_gather(slot, j, b, off):
        pltpu.make_async_copy(x0.at[sidx.at[slot, j]], xb[b], sg[b]).wait()
        pltpu.make_async_copy(e0.at[pl.ds(off, C)], eb[b], se[b]).wait()

    def wait_scatter(slot, j, b):
        pltpu.make_async_copy(xb[b], agg_sh.at[didx.at[slot, j]], ss[b]).wait()

    r0 = s * _SC_RPT
    pltpu.sync_copy(zrows, agg_sh.at[pl.ds(r0, _SC_RPT)])

    ebase = wid * ept
    pltpu.sync_copy(srcr.at[idx_rows(0)], sidx.at[0])
    pltpu.sync_copy(dstr.at[idx_rows(0)], didx.at[0])
    issue_gather(0, 0, 0, ebase)
    plsc.subcore_barrier()

    @pl.loop(0, M)
    def _super(m):
        mb = lax.rem(m, 2)
        for j in range(S):
            k = m * S + j
            b = j % 2
            off = ebase + k * C
            wait_gather(mb, j, b, off)
            _sc_relu_add(xb[b], eb[b], C)
            @pl.when(k >= 1)
            def _():
                if j == 0:
                    wait_scatter(1 - mb, S - 1, 1 - b)
                else:
                    wait_scatter(mb, j - 1, 1 - b)

            pltpu.async_copy(xb[b], agg_sh.at[didx.at[mb, j]], ss[b], add=True)

            if j == 0:
                @pl.when(m < M - 1)
                def _():
                    issue_pre(m + 1, 1 - mb)
            if j == S - 1:
                @pl.when(m < M - 1)
                def _():
                    wait_pre(m + 1, 1 - mb)
                    issue_gather(1 - mb, 0, 1 - b, off + C)
            else:
                @pl.when(k < n - 1)
                def _():
                    issue_gather(mb, j + 1, 1 - b, off + C)

    wait_scatter(lax.rem(M - 1, 2), S - 1, (S - 1) % 2)
    plsc.subcore_barrier()

    for j in range(_SC_RPT // _SC_WC):
        sl = pl.ds(r0 + j * _SC_WC, _SC_WC)

        @pl.when(c == 0)
        def _():
            pltpu.sync_copy(agg_sh.at[sl], out0.at[sl])

        @pl.when(c == 1)
        def _():
            pltpu.sync_copy(agg_sh.at[sl], out1.at[sl])


def _sc_kernel(C, S, edge_split):
    mesh = plsc.VectorSubcoreMesh(core_axis_name="c", subcore_axis_name="s",
                                  num_cores=2, num_subcores=16)
    scratch = [
        pltpu.VMEM_SHARED((NP, 128), jnp.float32),
        pltpu.VMEM((2, S, C), jnp.int32),
        pltpu.VMEM((2, S, C), jnp.int32),
        pltpu.VMEM((C, 128), jnp.float32),
        pltpu.VMEM((C, 128), jnp.float32),
        pltpu.VMEM((C, 128), jnp.float32),
        pltpu.VMEM((C, 128), jnp.float32),
    ] + [pltpu.SemaphoreType.DMA] * 7
    body = functools.partial(_sc_body, C=C, S=S, edge_split=edge_split)
    if edge_split:
        def body2(x0, srcr, dstr, e0, zrows, out0, out1,
                  agg_sh, sidx, didx, xb0, xb1, eb0, eb1,
                  sg0, sg1, se0, se1, ss0, ss1, spre):
            return body(x0, None, srcr, dstr, e0, None, zrows, out0, out1,
                        agg_sh, sidx, didx, xb0, xb1, eb0, eb1,
                        sg0, sg1, se0, se1, ss0, ss1, spre)
        use_body = body2
    else:
        use_body = body
    return pl.kernel(
        use_body,
        out_type=[jax.ShapeDtypeStruct((NP, 128), jnp.float32),
                  jax.ShapeDtypeStruct((NP, 128), jnp.float32)],
        mesh=mesh,
        scratch_types=scratch,
    )


@functools.cache
def _sc_msg():
    return _sc_kernel(C=80, S=8, edge_split=False)


@functools.cache
def _sc_msg0():
    return _sc_kernel(C=40, S=8, edge_split=True)


def _mlp_body(x0_ref, x1_ref, a0_ref, a1_ref, w1_ref, b1_ref, w2_ref, b2_ref,
              h2_ref, st_ref, *, B, layer0):
    if layer0:
        h = x0_ref[...] + a0_ref[...] + a1_ref[...]
    else:
        h = jnp.concatenate(
            [x0_ref[...] + a0_ref[...], x1_ref[...] + a1_ref[...]], axis=1)
    h1 = jnp.maximum(
        jnp.dot(h, w1_ref[...], preferred_element_type=jnp.float32) + b1_ref[...], 0.0)
    h2 = jnp.dot(h1, w2_ref[...], preferred_element_type=jnp.float32) + b2_ref[...]
    h2_ref[...] = h2
    gid = pl.program_id(0)
    rows = gid * B + jax.lax.broadcasted_iota(jnp.int32, (B, 1), 0)
    hm = h2 * (rows < N).astype(jnp.float32)

    @pl.when(gid == 0)
    def _():
        st_ref[...] = jnp.zeros_like(st_ref)

    st_ref[0:1, :] += jnp.sum(hm, axis=0, keepdims=True)
    st_ref[1:2, :] += jnp.sum(hm * hm, axis=0, keepdims=True)


def _node_mlp(x0, x1, a0, a1, w1p, b1, w2, b2, layer0):
    B = 512
    din = 128 if layer0 else 256
    return pl.pallas_call(
        functools.partial(_mlp_body, B=B, layer0=layer0),
        grid=(NP // B,),
        in_specs=[
            pl.BlockSpec((B, 128), lambda i: (i, 0)),
            pl.BlockSpec((B, 128), lambda i: (i, 0)),
            pl.BlockSpec((B, 128), lambda i: (i, 0)),
            pl.BlockSpec((B, 128), lambda i: (i, 0)),
            pl.BlockSpec((din, HID), lambda i: (0, 0)),
            pl.BlockSpec((1, HID), lambda i: (0, 0)),
            pl.BlockSpec((HID, HID), lambda i: (0, 0)),
            pl.BlockSpec((1, HID), lambda i: (0, 0)),
        ],
        out_specs=[
            pl.BlockSpec((B, HID), lambda i: (i, 0)),
            pl.BlockSpec((8, HID), lambda i: (0, 0)),
        ],
        out_shape=[
            jax.ShapeDtypeStruct((NP, HID), jnp.float32),
            jax.ShapeDtypeStruct((8, HID), jnp.float32),
        ],
    )(x0, x1, a0, a1, w1p, b1, w2, b2)


def _bn_body(h2_ref, st_ref, g_ref, bt_ref, y0_ref, y1_ref, pool_ref, *, B):
    inv_n = 1.0 / N
    mean = st_ref[0:1, :] * inv_n
    var = st_ref[1:2, :] * inv_n - mean * mean
    inv = jax.lax.rsqrt(var + 1e-5)
    y = jnp.maximum((h2_ref[...] - mean) * inv * g_ref[...] + bt_ref[...], 0.0)
    y0_ref[...] = y[:, :128]
    y1_ref[...] = y[:, 128:]
    gid = pl.program_id(0)
    rows = gid * B + jax.lax.broadcasted_iota(jnp.int32, (B, 1), 0)
    ym = y * (rows < N).astype(jnp.float32)

    @pl.when(gid == 0)
    def _():
        pool_ref[...] = jnp.zeros_like(pool_ref)

    pool_ref[0:1, :] += jnp.sum(ym, axis=0, keepdims=True)


def _bn_apply(h2, st, gamma, beta):
    B = 512
    return pl.pallas_call(
        functools.partial(_bn_body, B=B),
        grid=(NP // B,),
        in_specs=[
            pl.BlockSpec((B, HID), lambda i: (i, 0)),
            pl.BlockSpec((8, HID), lambda i: (0, 0)),
            pl.BlockSpec((1, HID), lambda i: (0, 0)),
            pl.BlockSpec((1, HID), lambda i: (0, 0)),
        ],
        out_specs=[
            pl.BlockSpec((B, 128), lambda i: (i, 0)),
            pl.BlockSpec((B, 128), lambda i: (i, 0)),
            pl.BlockSpec((8, HID), lambda i: (0, 0)),
        ],
        out_shape=[
            jax.ShapeDtypeStruct((NP, 128), jnp.float32),
            jax.ShapeDtypeStruct((NP, 128), jnp.float32),
            jax.ShapeDtypeStruct((8, HID), jnp.float32),
        ],
    )(h2, st, gamma, beta)


def _head_body(pool_ref, wm1_ref, bm1_ref, wm2_ref, bm2_ref, out_ref):
    g = pool_ref[...] * (1.0 / N)
    a = jnp.maximum(
        jnp.dot(g, wm1_ref[...], preferred_element_type=jnp.float32) + bm1_ref[...], 0.0)
    out_ref[...] = jnp.dot(a, wm2_ref[...], preferred_element_type=jnp.float32) + bm2_ref[...]


def _head(pool, wm1, bm1, wm2p, bm2p):
    return pl.pallas_call(
        _head_body,
        grid=(1,),
        in_specs=[
            pl.BlockSpec((8, HID), lambda i: (0, 0)),
            pl.BlockSpec((HID, HID), lambda i: (0, 0)),
            pl.BlockSpec((1, HID), lambda i: (0, 0)),
            pl.BlockSpec((HID, 128), lambda i: (0, 0)),
            pl.BlockSpec((1, 128), lambda i: (0, 0)),
        ],
        out_specs=pl.BlockSpec((8, 128), lambda i: (0, 0)),
        out_shape=jax.ShapeDtypeStruct((8, 128), jnp.float32),
    )(pool, wm1, bm1, wm2p, bm2p)


def kernel(atomic_number, other_feats, edge_index, edge_attr, params):
    layers = params["layers"]
    src = edge_index[0].astype(jnp.int32)
    dst = edge_index[1].astype(jnp.int32)

    an2d = jnp.pad(atomic_number.astype(jnp.int32), (0, NP - N)).reshape(NP, 1)
    of_pad = jnp.pad(other_feats, ((0, NP - N), (0, 0)))
    emb_pad = jnp.pad(params["emb"], ((0, NUM_AT_PAD - 100), (0, 0)))
    w0p = jnp.pad(layers[0]["We"], ((0, 0), (0, 128 - 72)))
    wcat = jnp.concatenate([w0p] + [layers[i]["We"] for i in (1, 2, 3)], axis=1)

    x0 = _embed(an2d, of_pad, emb_pad)
    x1 = None
    ea_pad = jnp.pad(edge_attr, ((0, EP - E), (0, 0)))
    e_parts = None

    zrows = jnp.zeros((_SC_RPT, 128), jnp.float32)
    srcp = jnp.pad(src, (0, EP - E))
    dstp = jnp.pad(dst, (0, EP - E), constant_values=N + 8)
    perm = jnp.argsort(dstp, stable=True)
    srcp = srcp[perm]
    dstp = dstp[perm]
    e_parts = _edge_mlp(ea_pad[perm], wcat)
    src80 = srcp.reshape(EP // 80, 80)
    dst80 = dstp.reshape(EP // 80, 80)
    src40 = srcp.reshape(EP // 40, 40)
    dst40 = dstp.reshape(EP // 40, 40)
    pool = None
    for li, l in enumerate(layers):
        if li == 0:
            a0, a1 = _sc_msg0()(x0, src40, dst40, e_parts[0], zrows)
        else:
            a0, a1 = _sc_msg()(x0, x1, src80, dst80,
                               e_parts[2 * li - 1], e_parts[2 * li], zrows)

        din = 72 if li == 0 else HID
        din_pad = 128 if li == 0 else HID
        w1p = jnp.pad(l["W1"], ((0, din_pad - din), (0, 0)))
        h2, st = _node_mlp(x0, x0 if x1 is None else x1, a0, a1, w1p,
                           l["b1"].reshape(1, HID), l["W2"],
                           l["b2"].reshape(1, HID), li == 0)
        x0, x1, pool = _bn_apply(h2, st, l["gamma"].reshape(1, HID),
                                 l["beta"].reshape(1, HID))

    wm2p = jnp.pad(params["Wm2"], ((0, 0), (0, 127)))
    bm2p = jnp.pad(params["bm2"].reshape(1, 1), ((0, 0), (0, 127)))
    out = _head(pool, params["Wm1"], params["bm1"].reshape(1, HID), wm2p, bm2p)
    return out[0, 0].reshape(1)


# speedup vs baseline: 1.3343x; 1.3343x over previous
import functools

import jax
import jax.numpy as jnp
from jax import lax
from jax.experimental import pallas as pl
from jax.experimental.pallas import tpu as pltpu
from jax.experimental.pallas import tpu_sc as plsc

N = 10000
NP = 10240
E = 320000
EP = 327680
HID = 256
ED = 16
NUM_AT_PAD = 104


def _embed_body(an_ref, of_ref, emb_ref, x0_ref):
    an = an_ref[...]
    ids = jax.lax.broadcasted_iota(jnp.int32, (1, NUM_AT_PAD), 1)
    oh = (an == ids).astype(jnp.float32)
    emb = jnp.dot(oh, emb_ref[...], preferred_element_type=jnp.float32)
    B = emb.shape[0]
    x0_ref[...] = jnp.concatenate(
        [emb, of_ref[...], jnp.zeros((B, 56), jnp.float32)], axis=1)


def _embed(an2d, of_pad, emb_pad):
    B = 1024
    return pl.pallas_call(
        _embed_body,
        grid=(NP // B,),
        in_specs=[
            pl.BlockSpec((B, 1), lambda i: (i, 0)),
            pl.BlockSpec((B, 8), lambda i: (i, 0)),
            pl.BlockSpec((NUM_AT_PAD, 64), lambda i: (0, 0)),
        ],
        out_specs=pl.BlockSpec((B, 128), lambda i: (i, 0)),
        out_shape=jax.ShapeDtypeStruct((NP, 128), jnp.float32),
    )(an2d, of_pad, emb_pad)


def _edge_mlp_body(ea_ref, w_ref, *out_refs):
    e = jnp.dot(ea_ref[...], w_ref[...], preferred_element_type=jnp.float32)
    for k, r in enumerate(out_refs):
        r[...] = e[:, 128 * k:128 * (k + 1)]


def _edge_mlp(edge_attr, wcat):
    B = 2048
    return pl.pallas_call(
        _edge_mlp_body,
        grid=(EP // B,),
        in_specs=[
            pl.BlockSpec((B, ED), lambda i: (i, 0)),
            pl.BlockSpec((ED, 896), lambda i: (0, 0)),
        ],
        out_specs=[pl.BlockSpec((B, 128), lambda i: (i, 0)) for _ in range(7)],
        out_shape=[jax.ShapeDtypeStruct((EP, 128), jnp.float32)
                   for _ in range(7)],
    )(edge_attr, wcat)


_SC_RPT = NP // 16
_SC_WC = 80


def _sc_relu_add(xbuf, ebuf, C):
    @pl.loop(0, C, unroll=2)
    def _edge(i):
        for j in range(8):
            sl = pl.ds(j * 16, 16)
            xbuf[i, sl] = jnp.maximum(xbuf[i, sl] + ebuf[i, sl], 0.0)


def _sc_body(x0, x1, srcr, dstr, e0, e1, zrows, out0, out1,
             agg_sh, sidx, didx, xb0, xb1, eb0, eb1,
             sg0, sg1, se0, se1, ss0, ss1, spre,
             *, C, S, edge_split):
    c = lax.axis_index("c")
    s = lax.axis_index("s")
    xb = (xb0, xb1)
    eb = (eb0, eb1)
    sg = (sg0, sg1)
    se = (se0, se1)
    ss = (ss0, ss1)

    nworkers = 32 if edge_split else 16
    ept = EP // nworkers
    n = ept // C
    M = n // S
    rows_pt = ept // C
    wid = (c * 16 + s) if edge_split else s
    rbase = wid * rows_pt

    def idx_rows(m):
        return pl.ds(rbase + m * S, S)

    def issue_pre(m, slot):
        pltpu.async_copy(srcr.at[idx_rows(m)], sidx.at[slot], spre)
        pltpu.async_copy(dstr.at[idx_rows(m)], didx.at[slot], spre)

    def wait_pre(m, slot):
        pltpu.make_async_copy(srcr.at[idx_rows(m)], sidx.at[slot], spre).wait()
        pltpu.make_async_copy(dstr.at[idx_rows(m)], didx.at[slot], spre).wait()

    def issue_gather(slot, j, b, off):
        if edge_split:
            pltpu.async_copy(x0.at[sidx.at[slot, j]], xb[b], sg[b])
            pltpu.async_copy(e0.at[pl.ds(off, C)], eb[b], se[b])
        else:
            @pl.when(c == 0)
            def _():
                pltpu.async_copy(x0.at[sidx.at[slot, j]], xb[b], sg[b])
                pltpu.async_copy(e0.at[pl.ds(off, C)], eb[b], se[b])

            @pl.when(c == 1)
            def _():
                pltpu.async_copy(x1.at[sidx.at[slot, j]], xb[b], sg[b])
                pltpu.async_copy(e1.at[pl.ds(off, C)], eb[b], se[b])

    def wait_gather(slot, j, b, off):
        pltpu.make_async_copy(x0.at[sidx.at[slot, j]], xb[b], sg[b]).wait()
        pltpu.make_async_copy(e0.at[pl.ds(off, C)], eb[b], se[b]).wait()

    def wait_scatter(slot, j, b):
        pltpu.make_async_copy(xb[b], agg_sh.at[didx.at[slot, j]], ss[b]).wait()

    r0 = s * _SC_RPT
    pltpu.sync_copy(zrows, agg_sh.at[pl.ds(r0, _SC_RPT)])

    ebase = wid * ept
    pltpu.sync_copy(srcr.at[idx_rows(0)], sidx.at[0])
    pltpu.sync_copy(dstr.at[idx_rows(0)], didx.at[0])
    issue_gather(0, 0, 0, ebase)
    plsc.subcore_barrier()

    @pl.loop(0, M)
    def _super(m):
        mb = lax.rem(m, 2)
        for j in range(S):
            k = m * S + j
            b = j % 2
            off = ebase + k * C
            wait_gather(mb, j, b, off)

            @pl.when(k >= 1)
            def _():
                if j == 0:
                    wait_scatter(1 - mb, S - 1, 1 - b)
                else:
                    wait_scatter(mb, j - 1, 1 - b)

            if j == S - 1:
                @pl.when(m < M - 1)
                def _():
                    wait_pre(m + 1, 1 - mb)
                    issue_gather(1 - mb, 0, 1 - b, off + C)
            else:
                @pl.when(k < n - 1)
                def _():
                    issue_gather(mb, j + 1, 1 - b, off + C)

            _sc_relu_add(xb[b], eb[b], C)
            pltpu.async_copy(xb[b], agg_sh.at[didx.at[mb, j]], ss[b], add=True)

            if j == 0:
                @pl.when(m < M - 1)
                def _():
                    issue_pre(m + 1, 1 - mb)

    wait_scatter(lax.rem(M - 1, 2), S - 1, (S - 1) % 2)
    plsc.subcore_barrier()

    for j in range(_SC_RPT // _SC_WC):
        sl = pl.ds(r0 + j * _SC_WC, _SC_WC)

        @pl.when(c == 0)
        def _():
            pltpu.sync_copy(agg_sh.at[sl], out0.at[sl])

        @pl.when(c == 1)
        def _():
            pltpu.sync_copy(agg_sh.at[sl], out1.at[sl])


def _sc_kernel(C, S, edge_split):
    mesh = plsc.VectorSubcoreMesh(core_axis_name="c", subcore_axis_name="s",
                                  num_cores=2, num_subcores=16)
    scratch = [
        pltpu.VMEM_SHARED((NP, 128), jnp.float32),
        pltpu.VMEM((2, S, C), jnp.int32),
        pltpu.VMEM((2, S, C), jnp.int32),
        pltpu.VMEM((C, 128), jnp.float32),
        pltpu.VMEM((C, 128), jnp.float32),
        pltpu.VMEM((C, 128), jnp.float32),
        pltpu.VMEM((C, 128), jnp.float32),
    ] + [pltpu.SemaphoreType.DMA] * 7
    body = functools.partial(_sc_body, C=C, S=S, edge_split=edge_split)
    if edge_split:
        def body2(x0, srcr, dstr, e0, zrows, out0, out1,
                  agg_sh, sidx, didx, xb0, xb1, eb0, eb1,
                  sg0, sg1, se0, se1, ss0, ss1, spre):
            return body(x0, None, srcr, dstr, e0, None, zrows, out0, out1,
                        agg_sh, sidx, didx, xb0, xb1, eb0, eb1,
                        sg0, sg1, se0, se1, ss0, ss1, spre)
        use_body = body2
    else:
        use_body = body
    return pl.kernel(
        use_body,
        out_type=[jax.ShapeDtypeStruct((NP, 128), jnp.float32),
                  jax.ShapeDtypeStruct((NP, 128), jnp.float32)],
        mesh=mesh,
        scratch_types=scratch,
    )


@functools.cache
def _sc_msg():
    return _sc_kernel(C=80, S=8, edge_split=False)


@functools.cache
def _sc_msg0():
    return _sc_kernel(C=40, S=8, edge_split=True)


def _mlp_body(x0_ref, x1_ref, a0_ref, a1_ref, w1_ref, b1_ref, w2_ref, b2_ref,
              h2_ref, st_ref, *, B, layer0):
    if layer0:
        h = x0_ref[...] + a0_ref[...] + a1_ref[...]
    else:
        h = jnp.concatenate(
            [x0_ref[...] + a0_ref[...], x1_ref[...] + a1_ref[...]], axis=1)
    h1 = jnp.maximum(
        jnp.dot(h, w1_ref[...], preferred_element_type=jnp.float32) + b1_ref[...], 0.0)
    h2 = jnp.dot(h1, w2_ref[...], preferred_element_type=jnp.float32) + b2_ref[...]
    h2_ref[...] = h2
    gid = pl.program_id(0)
    rows = gid * B + jax.lax.broadcasted_iota(jnp.int32, (B, 1), 0)
    hm = h2 * (rows < N).astype(jnp.float32)

    @pl.when(gid == 0)
    def _():
        st_ref[...] = jnp.zeros_like(st_ref)

    st_ref[0:1, :] += jnp.sum(hm, axis=0, keepdims=True)
    st_ref[1:2, :] += jnp.sum(hm * hm, axis=0, keepdims=True)


def _node_mlp(x0, x1, a0, a1, w1p, b1, w2, b2, layer0):
    B = 512
    din = 128 if layer0 else 256
    return pl.pallas_call(
        functools.partial(_mlp_body, B=B, layer0=layer0),
        grid=(NP // B,),
        in_specs=[
            pl.BlockSpec((B, 128), lambda i: (i, 0)),
            pl.BlockSpec((B, 128), lambda i: (i, 0)),
            pl.BlockSpec((B, 128), lambda i: (i, 0)),
            pl.BlockSpec((B, 128), lambda i: (i, 0)),
            pl.BlockSpec((din, HID), lambda i: (0, 0)),
            pl.BlockSpec((1, HID), lambda i: (0, 0)),
            pl.BlockSpec((HID, HID), lambda i: (0, 0)),
            pl.BlockSpec((1, HID), lambda i: (0, 0)),
        ],
        out_specs=[
            pl.BlockSpec((B, HID), lambda i: (i, 0)),
            pl.BlockSpec((8, HID), lambda i: (0, 0)),
        ],
        out_shape=[
            jax.ShapeDtypeStruct((NP, HID), jnp.float32),
            jax.ShapeDtypeStruct((8, HID), jnp.float32),
        ],
    )(x0, x1, a0, a1, w1p, b1, w2, b2)


def _bn_body(h2_ref, st_ref, g_ref, bt_ref, y0_ref, y1_ref, pool_ref, *, B):
    inv_n = 1.0 / N
    mean = st_ref[0:1, :] * inv_n
    var = st_ref[1:2, :] * inv_n - mean * mean
    inv = jax.lax.rsqrt(var + 1e-5)
    y = jnp.maximum((h2_ref[...] - mean) * inv * g_ref[...] + bt_ref[...], 0.0)
    y0_ref[...] = y[:, :128]
    y1_ref[...] = y[:, 128:]
    gid = pl.program_id(0)
    rows = gid * B + jax.lax.broadcasted_iota(jnp.int32, (B, 1), 0)
    ym = y * (rows < N).astype(jnp.float32)

    @pl.when(gid == 0)
    def _():
        pool_ref[...] = jnp.zeros_like(pool_ref)

    pool_ref[0:1, :] += jnp.sum(ym, axis=0, keepdims=True)


def _bn_apply(h2, st, gamma, beta):
    B = 512
    return pl.pallas_call(
        functools.partial(_bn_body, B=B),
        grid=(NP // B,),
        in_specs=[
            pl.BlockSpec((B, HID), lambda i: (i, 0)),
            pl.BlockSpec((8, HID), lambda i: (0, 0)),
            pl.BlockSpec((1, HID), lambda i: (0, 0)),
            pl.BlockSpec((1, HID), lambda i: (0, 0)),
        ],
        out_specs=[
            pl.BlockSpec((B, 128), lambda i: (i, 0)),
            pl.BlockSpec((B, 128), lambda i: (i, 0)),
            pl.BlockSpec((8, HID), lambda i: (0, 0)),
        ],
        out_shape=[
            jax.ShapeDtypeStruct((NP, 128), jnp.float32),
            jax.ShapeDtypeStruct((NP, 128), jnp.float32),
            jax.ShapeDtypeStruct((8, HID), jnp.float32),
        ],
    )(h2, st, gamma, beta)


def _head_body(pool_ref, wm1_ref, bm1_ref, wm2_ref, bm2_ref, out_ref):
    g = pool_ref[...] * (1.0 / N)
    a = jnp.maximum(
        jnp.dot(g, wm1_ref[...], preferred_element_type=jnp.float32) + bm1_ref[...], 0.0)
    out_ref[...] = jnp.dot(a, wm2_ref[...], preferred_element_type=jnp.float32) + bm2_ref[...]


def _head(pool, wm1, bm1, wm2p, bm2p):
    return pl.pallas_call(
        _head_body,
        grid=(1,),
        in_specs=[
            pl.BlockSpec((8, HID), lambda i: (0, 0)),
            pl.BlockSpec((HID, HID), lambda i: (0, 0)),
            pl.BlockSpec((1, HID), lambda i: (0, 0)),
            pl.BlockSpec((HID, 128), lambda i: (0, 0)),
            pl.BlockSpec((1, 128), lambda i: (0, 0)),
        ],
        out_specs=pl.BlockSpec((8, 128), lambda i: (0, 0)),
        out_shape=jax.ShapeDtypeStruct((8, 128), jnp.float32),
    )(pool, wm1, bm1, wm2p, bm2p)


def kernel(atomic_number, other_feats, edge_index, edge_attr, params):
    layers = params["layers"]
    src = edge_index[0].astype(jnp.int32)
    dst = edge_index[1].astype(jnp.int32)

    an2d = jnp.pad(atomic_number.astype(jnp.int32), (0, NP - N)).reshape(NP, 1)
    of_pad = jnp.pad(other_feats, ((0, NP - N), (0, 0)))
    emb_pad = jnp.pad(params["emb"], ((0, NUM_AT_PAD - 100), (0, 0)))
    w0p = jnp.pad(layers[0]["We"], ((0, 0), (0, 128 - 72)))
    wcat = jnp.concatenate([w0p] + [layers[i]["We"] for i in (1, 2, 3)], axis=1)

    x0 = _embed(an2d, of_pad, emb_pad)
    x1 = None
    ea_pad = jnp.pad(edge_attr, ((0, EP - E), (0, 0)))
    e_parts = None

    zrows = jnp.zeros((_SC_RPT, 128), jnp.float32)
    srcp = jnp.pad(src, (0, EP - E))
    dstp = jnp.pad(dst, (0, EP - E), constant_values=N + 8)
    perm = jnp.argsort(dstp, stable=True)
    srcp = srcp[perm]
    dstp = dstp[perm]
    e_parts = _edge_mlp(ea_pad[perm], wcat)
    src80 = srcp.reshape(EP // 80, 80)
    dst80 = dstp.reshape(EP // 80, 80)
    src40 = srcp.reshape(EP // 40, 40)
    dst40 = dstp.reshape(EP // 40, 40)
    pool = None
    for li, l in enumerate(layers):
        if li == 0:
            a0, a1 = _sc_msg0()(x0, src40, dst40, e_parts[0], zrows)
        else:
            a0, a1 = _sc_msg()(x0, x1, src80, dst80,
                               e_parts[2 * li - 1], e_parts[2 * li], zrows)

        din = 72 if li == 0 else HID
        din_pad = 128 if li == 0 else HID
        w1p = jnp.pad(l["W1"], ((0, din_pad - din), (0, 0)))
        h2, st = _node_mlp(x0, x0 if x1 is None else x1, a0, a1, w1p,
                           l["b1"].reshape(1, HID), l["W2"],
                           l["b2"].reshape(1, HID), li == 0)
        x0, x1, pool = _bn_apply(h2, st, l["gamma"].reshape(1, HID),
                                 l["beta"].reshape(1, HID))

    wm2p = jnp.pad(params["Wm2"], ((0, 0), (0, 127)))
    bm2p = jnp.pad(params["bm2"].reshape(1, 1), ((0, 0), (0, 127)))
    out = _head(pool, params["Wm1"], params["bm1"].reshape(1, HID), wm2p, bm2p)
    return out[0, 0].reshape(1)


# speedup vs baseline: 1.3994x; 1.0487x over previous
import functools

import jax
import jax.numpy as jnp
from jax import lax
from jax.experimental import pallas as pl
from jax.experimental.pallas import tpu as pltpu
from jax.experimental.pallas import tpu_sc as plsc

N = 10000
NP = 10240
E = 320000
EP = 327680
HID = 256
ED = 16
NUM_AT_PAD = 104


def _embed_body(an_ref, of_ref, emb_ref, x0_ref):
    an = an_ref[...]
    ids = jax.lax.broadcasted_iota(jnp.int32, (1, NUM_AT_PAD), 1)
    oh = (an == ids).astype(jnp.float32)
    emb = jnp.dot(oh, emb_ref[...], preferred_element_type=jnp.float32)
    B = emb.shape[0]
    x0_ref[...] = jnp.concatenate(
        [emb, of_ref[...], jnp.zeros((B, 56), jnp.float32)], axis=1)


def _embed(an2d, of_pad, emb_pad):
    B = 1024
    return pl.pallas_call(
        _embed_body,
        grid=(NP // B,),
        in_specs=[
            pl.BlockSpec((B, 1), lambda i: (i, 0)),
            pl.BlockSpec((B, 8), lambda i: (i, 0)),
            pl.BlockSpec((NUM_AT_PAD, 64), lambda i: (0, 0)),
        ],
        out_specs=pl.BlockSpec((B, 128), lambda i: (i, 0)),
        out_shape=jax.ShapeDtypeStruct((NP, 128), jnp.float32),
    )(an2d, of_pad, emb_pad)


def _edge_mlp_body(ea_ref, w_ref, *out_refs):
    e = jnp.dot(ea_ref[...], w_ref[...], preferred_element_type=jnp.float32)
    for k, r in enumerate(out_refs):
        r[...] = e[:, 128 * k:128 * (k + 1)]


def _edge_mlp(edge_attr, wcat):
    B = 2048
    return pl.pallas_call(
        _edge_mlp_body,
        grid=(EP // B,),
        in_specs=[
            pl.BlockSpec((B, ED), lambda i: (i, 0)),
            pl.BlockSpec((ED, 896), lambda i: (0, 0)),
        ],
        out_specs=[pl.BlockSpec((B, 128), lambda i: (i, 0)) for _ in range(7)],
        out_shape=[jax.ShapeDtypeStruct((EP, 128), jnp.float32)
                   for _ in range(7)],
    )(edge_attr, wcat)


_SC_RPT = NP // 16
_SC_WC = 80


def _sc_relu_add(xbuf, ebuf, C):
    @pl.loop(0, C, unroll=2)
    def _edge(i):
        for j in range(8):
            sl = pl.ds(j * 16, 16)
            xbuf[i, sl] = jnp.maximum(xbuf[i, sl] + ebuf[i, sl], 0.0)


def _sc_body(x0, x1, srcr, dstr, e0, e1, zrows, out0, out1,
             agg_sh, sidx, didx, xb0, xb1, xb2, xb3, eb0, eb1,
             sg0, sg1, sg2, sg3, se0, se1, ss0, ss1, ss2, ss3, spre,
             *, C, S, edge_split):
    c = lax.axis_index("c")
    s = lax.axis_index("s")
    xb = (xb0, xb1, xb2, xb3)
    eb = (eb0, eb1)
    sg = (sg0, sg1, sg2, sg3)
    se = (se0, se1)
    ss = (ss0, ss1, ss2, ss3)

    nworkers = 32 if edge_split else 16
    ept = EP // nworkers
    n = ept // C
    M = n // S
    rows_pt = ept // C
    wid = (c * 16 + s) if edge_split else s
    rbase = wid * rows_pt

    def idx_rows(m):
        return pl.ds(rbase + m * S, S)

    def issue_pre(m, slot):
        pltpu.async_copy(srcr.at[idx_rows(m)], sidx.at[slot], spre)
        pltpu.async_copy(dstr.at[idx_rows(m)], didx.at[slot], spre)

    def wait_pre(m, slot):
        pltpu.make_async_copy(srcr.at[idx_rows(m)], sidx.at[slot], spre).wait()
        pltpu.make_async_copy(dstr.at[idx_rows(m)], didx.at[slot], spre).wait()

    def issue_gather(slot, j, b4, b2, off):
        if edge_split:
            pltpu.async_copy(x0.at[sidx.at[slot, j]], xb[b4], sg[b4])
            pltpu.async_copy(e0.at[pl.ds(off, C)], eb[b2], se[b2])
        else:
            @pl.when(c == 0)
            def _():
                pltpu.async_copy(x0.at[sidx.at[slot, j]], xb[b4], sg[b4])
                pltpu.async_copy(e0.at[pl.ds(off, C)], eb[b2], se[b2])

            @pl.when(c == 1)
            def _():
                pltpu.async_copy(x1.at[sidx.at[slot, j]], xb[b4], sg[b4])
                pltpu.async_copy(e1.at[pl.ds(off, C)], eb[b2], se[b2])

    def wait_gather(slot, j, b4, b2, off):
        pltpu.make_async_copy(x0.at[sidx.at[slot, j]], xb[b4], sg[b4]).wait()
        pltpu.make_async_copy(e0.at[pl.ds(off, C)], eb[b2], se[b2]).wait()

    def wait_scatter(slot, j, b4):
        pltpu.make_async_copy(xb[b4], agg_sh.at[didx.at[slot, j]], ss[b4]).wait()

    r0 = s * _SC_RPT
    pltpu.sync_copy(zrows, agg_sh.at[pl.ds(r0, _SC_RPT)])

    ebase = wid * ept
    pltpu.sync_copy(srcr.at[idx_rows(0)], sidx.at[0])
    pltpu.sync_copy(dstr.at[idx_rows(0)], didx.at[0])
    issue_gather(0, 0, 0, 0, ebase)
    plsc.subcore_barrier()

    @pl.loop(0, M)
    def _super(m):
        mb = lax.rem(m, 2)
        for j in range(S):
            k = m * S + j
            b4 = j % 4
            b2 = j % 2
            off = ebase + k * C
            wait_gather(mb, j, b4, b2, off)

            @pl.when(k >= 3)
            def _():
                if j >= 3:
                    wait_scatter(mb, j - 3, (j - 3) % 4)
                else:
                    wait_scatter(1 - mb, S - 3 + j, (S - 3 + j) % 4)

            if j == S - 1:
                @pl.when(m < M - 1)
                def _():
                    wait_pre(m + 1, 1 - mb)
                    issue_gather(1 - mb, 0, 0, 0, off + C)
            else:
                @pl.when(k < n - 1)
                def _():
                    issue_gather(mb, j + 1, (j + 1) % 4, (j + 1) % 2, off + C)

            _sc_relu_add(xb[b4], eb[b2], C)
            pltpu.async_copy(xb[b4], agg_sh.at[didx.at[mb, j]], ss[b4], add=True)

            if j == 3:
                @pl.when(m < M - 1)
                def _():
                    issue_pre(m + 1, 1 - mb)

    for jj in (S - 3, S - 2, S - 1):
        wait_scatter(lax.rem(M - 1, 2), jj, jj % 4)
    plsc.subcore_barrier()

    for j in range(_SC_RPT // _SC_WC):
        sl = pl.ds(r0 + j * _SC_WC, _SC_WC)

        @pl.when(c == 0)
        def _():
            pltpu.sync_copy(agg_sh.at[sl], out0.at[sl])

        @pl.when(c == 1)
        def _():
            pltpu.sync_copy(agg_sh.at[sl], out1.at[sl])


def _sc_kernel(C, S, edge_split):
    mesh = plsc.VectorSubcoreMesh(core_axis_name="c", subcore_axis_name="s",
                                  num_cores=2, num_subcores=16)
    scratch = [
        pltpu.VMEM_SHARED((NP, 128), jnp.float32),
        pltpu.VMEM((2, S, C), jnp.int32),
        pltpu.VMEM((2, S, C), jnp.int32),
        pltpu.VMEM((C, 128), jnp.float32),
        pltpu.VMEM((C, 128), jnp.float32),
        pltpu.VMEM((C, 128), jnp.float32),
        pltpu.VMEM((C, 128), jnp.float32),
        pltpu.VMEM((C, 128), jnp.float32),
        pltpu.VMEM((C, 128), jnp.float32),
    ] + [pltpu.SemaphoreType.DMA] * 11
    body = functools.partial(_sc_body, C=C, S=S, edge_split=edge_split)
    if edge_split:
        def body2(x0, srcr, dstr, e0, zrows, out0, out1,
                  agg_sh, sidx, didx, xb0, xb1, xb2, xb3, eb0, eb1,
                  sg0, sg1, sg2, sg3, se0, se1, ss0, ss1, ss2, ss3, spre):
            return body(x0, None, srcr, dstr, e0, None, zrows, out0, out1,
                        agg_sh, sidx, didx, xb0, xb1, xb2, xb3, eb0, eb1,
                        sg0, sg1, sg2, sg3, se0, se1, ss0, ss1, ss2, ss3, spre)
        use_body = body2
    else:
        use_body = body
    return pl.kernel(
        use_body,
        out_type=[jax.ShapeDtypeStruct((NP, 128), jnp.float32),
                  jax.ShapeDtypeStruct((NP, 128), jnp.float32)],
        mesh=mesh,
        scratch_types=scratch,
    )


@functools.cache
def _sc_msg():
    return _sc_kernel(C=40, S=8, edge_split=False)


@functools.cache
def _sc_msg0():
    return _sc_kernel(C=40, S=8, edge_split=True)


def _mlp_body(x0_ref, x1_ref, a0_ref, a1_ref, w1_ref, b1_ref, w2_ref, b2_ref,
              h2_ref, st_ref, *, B, layer0):
    if layer0:
        h = x0_ref[...] + a0_ref[...] + a1_ref[...]
    else:
        h = jnp.concatenate(
            [x0_ref[...] + a0_ref[...], x1_ref[...] + a1_ref[...]], axis=1)
    h1 = jnp.maximum(
        jnp.dot(h, w1_ref[...], preferred_element_type=jnp.float32) + b1_ref[...], 0.0)
    h2 = jnp.dot(h1, w2_ref[...], preferred_element_type=jnp.float32) + b2_ref[...]
    h2_ref[...] = h2
    gid = pl.program_id(0)
    rows = gid * B + jax.lax.broadcasted_iota(jnp.int32, (B, 1), 0)
    hm = h2 * (rows < N).astype(jnp.float32)

    @pl.when(gid == 0)
    def _():
        st_ref[...] = jnp.zeros_like(st_ref)

    st_ref[0:1, :] += jnp.sum(hm, axis=0, keepdims=True)
    st_ref[1:2, :] += jnp.sum(hm * hm, axis=0, keepdims=True)


def _node_mlp(x0, x1, a0, a1, w1p, b1, w2, b2, layer0):
    B = 512
    din = 128 if layer0 else 256
    return pl.pallas_call(
        functools.partial(_mlp_body, B=B, layer0=layer0),
        grid=(NP // B,),
        in_specs=[
            pl.BlockSpec((B, 128), lambda i: (i, 0)),
            pl.BlockSpec((B, 128), lambda i: (i, 0)),
            pl.BlockSpec((B, 128), lambda i: (i, 0)),
            pl.BlockSpec((B, 128), lambda i: (i, 0)),
            pl.BlockSpec((din, HID), lambda i: (0, 0)),
            pl.BlockSpec((1, HID), lambda i: (0, 0)),
            pl.BlockSpec((HID, HID), lambda i: (0, 0)),
            pl.BlockSpec((1, HID), lambda i: (0, 0)),
        ],
        out_specs=[
            pl.BlockSpec((B, HID), lambda i: (i, 0)),
            pl.BlockSpec((8, HID), lambda i: (0, 0)),
        ],
        out_shape=[
            jax.ShapeDtypeStruct((NP, HID), jnp.float32),
            jax.ShapeDtypeStruct((8, HID), jnp.float32),
        ],
    )(x0, x1, a0, a1, w1p, b1, w2, b2)


def _bn_body(h2_ref, st_ref, g_ref, bt_ref, y0_ref, y1_ref, pool_ref, *, B):
    inv_n = 1.0 / N
    mean = st_ref[0:1, :] * inv_n
    var = st_ref[1:2, :] * inv_n - mean * mean
    inv = jax.lax.rsqrt(var + 1e-5)
    y = jnp.maximum((h2_ref[...] - mean) * inv * g_ref[...] + bt_ref[...], 0.0)
    y0_ref[...] = y[:, :128]
    y1_ref[...] = y[:, 128:]
    gid = pl.program_id(0)
    rows = gid * B + jax.lax.broadcasted_iota(jnp.int32, (B, 1), 0)
    ym = y * (rows < N).astype(jnp.float32)

    @pl.when(gid == 0)
    def _():
        pool_ref[...] = jnp.zeros_like(pool_ref)

    pool_ref[0:1, :] += jnp.sum(ym, axis=0, keepdims=True)


def _bn_apply(h2, st, gamma, beta):
    B = 512
    return pl.pallas_call(
        functools.partial(_bn_body, B=B),
        grid=(NP // B,),
        in_specs=[
            pl.BlockSpec((B, HID), lambda i: (i, 0)),
            pl.BlockSpec((8, HID), lambda i: (0, 0)),
            pl.BlockSpec((1, HID), lambda i: (0, 0)),
            pl.BlockSpec((1, HID), lambda i: (0, 0)),
        ],
        out_specs=[
            pl.BlockSpec((B, 128), lambda i: (i, 0)),
            pl.BlockSpec((B, 128), lambda i: (i, 0)),
            pl.BlockSpec((8, HID), lambda i: (0, 0)),
        ],
        out_shape=[
            jax.ShapeDtypeStruct((NP, 128), jnp.float32),
            jax.ShapeDtypeStruct((NP, 128), jnp.float32),
            jax.ShapeDtypeStruct((8, HID), jnp.float32),
        ],
    )(h2, st, gamma, beta)


def _head_body(pool_ref, wm1_ref, bm1_ref, wm2_ref, bm2_ref, out_ref):
    g = pool_ref[...] * (1.0 / N)
    a = jnp.maximum(
        jnp.dot(g, wm1_ref[...], preferred_element_type=jnp.float32) + bm1_ref[...], 0.0)
    out_ref[...] = jnp.dot(a, wm2_ref[...], preferred_element_type=jnp.float32) + bm2_ref[...]


def _head(pool, wm1, bm1, wm2p, bm2p):
    return pl.pallas_call(
        _head_body,
        grid=(1,),
        in_specs=[
            pl.BlockSpec((8, HID), lambda i: (0, 0)),
            pl.BlockSpec((HID, HID), lambda i: (0, 0)),
            pl.BlockSpec((1, HID), lambda i: (0, 0)),
            pl.BlockSpec((HID, 128), lambda i: (0, 0)),
            pl.BlockSpec((1, 128), lambda i: (0, 0)),
        ],
        out_specs=pl.BlockSpec((8, 128), lambda i: (0, 0)),
        out_shape=jax.ShapeDtypeStruct((8, 128), jnp.float32),
    )(pool, wm1, bm1, wm2p, bm2p)


def kernel(atomic_number, other_feats, edge_index, edge_attr, params):
    layers = params["layers"]
    src = edge_index[0].astype(jnp.int32)
    dst = edge_index[1].astype(jnp.int32)

    an2d = jnp.pad(atomic_number.astype(jnp.int32), (0, NP - N)).reshape(NP, 1)
    of_pad = jnp.pad(other_feats, ((0, NP - N), (0, 0)))
    emb_pad = jnp.pad(params["emb"], ((0, NUM_AT_PAD - 100), (0, 0)))
    w0p = jnp.pad(layers[0]["We"], ((0, 0), (0, 128 - 72)))
    wcat = jnp.concatenate([w0p] + [layers[i]["We"] for i in (1, 2, 3)], axis=1)

    x0 = _embed(an2d, of_pad, emb_pad)
    x1 = None
    ea_pad = jnp.pad(edge_attr, ((0, EP - E), (0, 0)))
    e_parts = None

    zrows = jnp.zeros((_SC_RPT, 128), jnp.float32)
    srcp = jnp.pad(src, (0, EP - E))
    dstp = jnp.pad(dst, (0, EP - E), constant_values=N + 8)
    perm = jnp.argsort(dstp, stable=True)
    srcp = srcp[perm]
    dstp = dstp[perm]
    e_parts = _edge_mlp(ea_pad[perm], wcat)
    srcr = srcp.reshape(EP // 40, 40)
    dstr = dstp.reshape(EP // 40, 40)
    pool = None
    for li, l in enumerate(layers):
        if li == 0:
            a0, a1 = _sc_msg0()(x0, srcr, dstr, e_parts[0], zrows)
        else:
            a0, a1 = _sc_msg()(x0, x1, srcr, dstr,
                               e_parts[2 * li - 1], e_parts[2 * li], zrows)

        din = 72 if li == 0 else HID
        din_pad = 128 if li == 0 else HID
        w1p = jnp.pad(l["W1"], ((0, din_pad - din), (0, 0)))
        h2, st = _node_mlp(x0, x0 if x1 is None else x1, a0, a1, w1p,
                           l["b1"].reshape(1, HID), l["W2"],
                           l["b2"].reshape(1, HID), li == 0)
        x0, x1, pool = _bn_apply(h2, st, l["gamma"].reshape(1, HID),
                                 l["beta"].reshape(1, HID))

    wm2p = jnp.pad(params["Wm2"], ((0, 0), (0, 127)))
    bm2p = jnp.pad(params["bm2"].reshape(1, 1), ((0, 0), (0, 127)))
    out = _head(pool, params["Wm1"], params["bm1"].reshape(1, HID), wm2p, bm2p)
    return out[0, 0].reshape(1)


# speedup vs baseline: 1.4814x; 1.0586x over previous
import functools

import jax
import jax.numpy as jnp
from jax import lax
from jax.experimental import pallas as pl
from jax.experimental.pallas import tpu as pltpu
from jax.experimental.pallas import tpu_sc as plsc

N = 10000
NP = 10240
E = 320000
EP = 327680
HID = 256
ED = 16
NUM_AT_PAD = 104


def _embed_body(an_ref, of_ref, emb_ref, x0_ref):
    an = an_ref[...]
    ids = jax.lax.broadcasted_iota(jnp.int32, (1, NUM_AT_PAD), 1)
    oh = (an == ids).astype(jnp.float32)
    emb = jnp.dot(oh, emb_ref[...], preferred_element_type=jnp.float32)
    B = emb.shape[0]
    x0_ref[...] = jnp.concatenate(
        [emb, of_ref[...], jnp.zeros((B, 56), jnp.float32)], axis=1)


def _embed(an2d, of_pad, emb_pad):
    B = 1024
    return pl.pallas_call(
        _embed_body,
        grid=(NP // B,),
        in_specs=[
            pl.BlockSpec((B, 1), lambda i: (i, 0)),
            pl.BlockSpec((B, 8), lambda i: (i, 0)),
            pl.BlockSpec((NUM_AT_PAD, 64), lambda i: (0, 0)),
        ],
        out_specs=pl.BlockSpec((B, 128), lambda i: (i, 0)),
        out_shape=jax.ShapeDtypeStruct((NP, 128), jnp.float32),
    )(an2d, of_pad, emb_pad)


def _edge_mlp_body(ea_ref, w_ref, *out_refs):
    e = jnp.dot(ea_ref[...], w_ref[...], preferred_element_type=jnp.float32)
    for k, r in enumerate(out_refs):
        r[...] = e[:, 128 * k:128 * (k + 1)]


def _edge_mlp(edge_attr, wcat):
    B = 2048
    return pl.pallas_call(
        _edge_mlp_body,
        grid=(EP // B,),
        in_specs=[
            pl.BlockSpec((B, ED), lambda i: (i, 0)),
            pl.BlockSpec((ED, 896), lambda i: (0, 0)),
        ],
        out_specs=[pl.BlockSpec((B, 128), lambda i: (i, 0)) for _ in range(7)],
        out_shape=[jax.ShapeDtypeStruct((EP, 128), jnp.float32)
                   for _ in range(7)],
    )(edge_attr, wcat)


_SC_RPT = NP // 16
_SC_WC = 80


def _sc_relu_add(xbuf, ebuf, C):
    @pl.loop(0, C, unroll=2)
    def _edge(i):
        for j in range(8):
            sl = pl.ds(j * 16, 16)
            xbuf[i, sl] = jnp.maximum(xbuf[i, sl] + ebuf[i, sl], 0.0)


def _sc_body(x0, x1, srcr, dstr, e0, e1, zrows, out0, out1,
             agg_sh, sidx, didx, xb0, xb1, xb2, xb3, eb0, eb1, eb2, eb3,
             sg0, sg1, sg2, sg3, se0, se1, se2, se3, ss0, ss1, ss2, ss3, spre,
             *, C, S, edge_split):
    c = lax.axis_index("c")
    s = lax.axis_index("s")
    xb = (xb0, xb1, xb2, xb3)
    eb = (eb0, eb1, eb2, eb3)
    sg = (sg0, sg1, sg2, sg3)
    se = (se0, se1, se2, se3)
    ss = (ss0, ss1, ss2, ss3)

    nworkers = 32 if edge_split else 16
    ept = EP // nworkers
    n = ept // C
    M = n // S
    rows_pt = ept // C
    wid = (c * 16 + s) if edge_split else s
    rbase = wid * rows_pt

    def idx_rows(m):
        return pl.ds(rbase + m * S, S)

    def issue_pre(m, slot):
        pltpu.async_copy(srcr.at[idx_rows(m)], sidx.at[slot], spre)
        pltpu.async_copy(dstr.at[idx_rows(m)], didx.at[slot], spre)

    def wait_pre(m, slot):
        pltpu.make_async_copy(srcr.at[idx_rows(m)], sidx.at[slot], spre).wait()
        pltpu.make_async_copy(dstr.at[idx_rows(m)], didx.at[slot], spre).wait()

    def issue_gather(slot, j, b4, off):
        if edge_split:
            pltpu.async_copy(x0.at[sidx.at[slot, j]], xb[b4], sg[b4])
            pltpu.async_copy(e0.at[pl.ds(off, C)], eb[b4], se[b4])
        else:
            @pl.when(c == 0)
            def _():
                pltpu.async_copy(x0.at[sidx.at[slot, j]], xb[b4], sg[b4])
                pltpu.async_copy(e0.at[pl.ds(off, C)], eb[b4], se[b4])

            @pl.when(c == 1)
            def _():
                pltpu.async_copy(x1.at[sidx.at[slot, j]], xb[b4], sg[b4])
                pltpu.async_copy(e1.at[pl.ds(off, C)], eb[b4], se[b4])

    def wait_gather(slot, j, b4, off):
        pltpu.make_async_copy(x0.at[sidx.at[slot, j]], xb[b4], sg[b4]).wait()
        pltpu.make_async_copy(e0.at[pl.ds(off, C)], eb[b4], se[b4]).wait()

    def wait_scatter(slot, j, b4):
        pltpu.make_async_copy(xb[b4], agg_sh.at[didx.at[slot, j]], ss[b4]).wait()

    r0 = s * _SC_RPT
    pltpu.sync_copy(zrows, agg_sh.at[pl.ds(r0, _SC_RPT)])

    ebase = wid * ept
    pltpu.sync_copy(srcr.at[idx_rows(0)], sidx.at[0])
    pltpu.sync_copy(dstr.at[idx_rows(0)], didx.at[0])
    issue_gather(0, 0, 0, ebase)
    issue_gather(0, 1, 1, ebase + C)
    plsc.subcore_barrier()

    @pl.loop(0, M)
    def _super(m):
        mb = lax.rem(m, 2)
        for j in range(S):
            k = m * S + j
            b4 = j % 4
            off = ebase + k * C
            wait_gather(mb, j, b4, off)

            @pl.when(k >= 2)
            def _():
                if j >= 2:
                    wait_scatter(mb, j - 2, (j - 2) % 4)
                else:
                    wait_scatter(1 - mb, S - 2 + j, (S - 2 + j) % 4)

            if j == S - 2:
                @pl.when(m < M - 1)
                def _():
                    wait_pre(m + 1, 1 - mb)
                    issue_gather(1 - mb, 0, (j + 2) % 4, off + 2 * C)
            elif j == S - 1:
                @pl.when(m < M - 1)
                def _():
                    issue_gather(1 - mb, 1, (j + 2) % 4, off + 2 * C)
            else:
                @pl.when(k < n - 2)
                def _():
                    issue_gather(mb, j + 2, (j + 2) % 4, off + 2 * C)

            _sc_relu_add(xb[b4], eb[b4], C)
            pltpu.async_copy(xb[b4], agg_sh.at[didx.at[mb, j]], ss[b4], add=True)

            if j == 3:
                @pl.when(m < M - 1)
                def _():
                    issue_pre(m + 1, 1 - mb)

    for jj in (S - 2, S - 1):
        wait_scatter(lax.rem(M - 1, 2), jj, jj % 4)
    plsc.subcore_barrier()

    for j in range(_SC_RPT // _SC_WC):
        sl = pl.ds(r0 + j * _SC_WC, _SC_WC)

        @pl.when(c == 0)
        def _():
            pltpu.sync_copy(agg_sh.at[sl], out0.at[sl])

        @pl.when(c == 1)
        def _():
            pltpu.sync_copy(agg_sh.at[sl], out1.at[sl])


def _sc_kernel(C, S, edge_split):
    mesh = plsc.VectorSubcoreMesh(core_axis_name="c", subcore_axis_name="s",
                                  num_cores=2, num_subcores=16)
    scratch = [
        pltpu.VMEM_SHARED((NP, 128), jnp.float32),
        pltpu.VMEM((2, S, C), jnp.int32),
        pltpu.VMEM((2, S, C), jnp.int32),
        pltpu.VMEM((C, 128), jnp.float32),
        pltpu.VMEM((C, 128), jnp.float32),
        pltpu.VMEM((C, 128), jnp.float32),
        pltpu.VMEM((C, 128), jnp.float32),
        pltpu.VMEM((C, 128), jnp.float32),
        pltpu.VMEM((C, 128), jnp.float32),
        pltpu.VMEM((C, 128), jnp.float32),
        pltpu.VMEM((C, 128), jnp.float32),
    ] + [pltpu.SemaphoreType.DMA] * 13
    body = functools.partial(_sc_body, C=C, S=S, edge_split=edge_split)
    if edge_split:
        def body2(x0, srcr, dstr, e0, zrows, out0, out1,
                  agg_sh, sidx, didx, xb0, xb1, xb2, xb3, eb0, eb1, eb2, eb3,
                  sg0, sg1, sg2, sg3, se0, se1, se2, se3,
                  ss0, ss1, ss2, ss3, spre):
            return body(x0, None, srcr, dstr, e0, None, zrows, out0, out1,
                        agg_sh, sidx, didx, xb0, xb1, xb2, xb3,
                        eb0, eb1, eb2, eb3, sg0, sg1, sg2, sg3,
                        se0, se1, se2, se3, ss0, ss1, ss2, ss3, spre)
        use_body = body2
    else:
        use_body = body
    return pl.kernel(
        use_body,
        out_type=[jax.ShapeDtypeStruct((NP, 128), jnp.float32),
                  jax.ShapeDtypeStruct((NP, 128), jnp.float32)],
        mesh=mesh,
        scratch_types=scratch,
    )


@functools.cache
def _sc_msg():
    return _sc_kernel(C=40, S=8, edge_split=False)


@functools.cache
def _sc_msg0():
    return _sc_kernel(C=40, S=8, edge_split=True)


def _mlp_body(x0_ref, x1_ref, a0_ref, a1_ref, w1_ref, b1_ref, w2_ref, b2_ref,
              h2_ref, st_ref, *, B, layer0):
    if layer0:
        h = x0_ref[...] + a0_ref[...] + a1_ref[...]
    else:
        h = jnp.concatenate(
            [x0_ref[...] + a0_ref[...], x1_ref[...] + a1_ref[...]], axis=1)
    h1 = jnp.maximum(
        jnp.dot(h, w1_ref[...], preferred_element_type=jnp.float32) + b1_ref[...], 0.0)
    h2 = jnp.dot(h1, w2_ref[...], preferred_element_type=jnp.float32) + b2_ref[...]
    h2_ref[...] = h2
    gid = pl.program_id(0)
    rows = gid * B + jax.lax.broadcasted_iota(jnp.int32, (B, 1), 0)
    hm = h2 * (rows < N).astype(jnp.float32)

    @pl.when(gid == 0)
    def _():
        st_ref[...] = jnp.zeros_like(st_ref)

    st_ref[0:1, :] += jnp.sum(hm, axis=0, keepdims=True)
    st_ref[1:2, :] += jnp.sum(hm * hm, axis=0, keepdims=True)


def _node_mlp(x0, x1, a0, a1, w1p, b1, w2, b2, layer0):
    B = 512
    din = 128 if layer0 else 256
    return pl.pallas_call(
        functools.partial(_mlp_body, B=B, layer0=layer0),
        grid=(NP // B,),
        in_specs=[
            pl.BlockSpec((B, 128), lambda i: (i, 0)),
            pl.BlockSpec((B, 128), lambda i: (i, 0)),
            pl.BlockSpec((B, 128), lambda i: (i, 0)),
            pl.BlockSpec((B, 128), lambda i: (i, 0)),
            pl.BlockSpec((din, HID), lambda i: (0, 0)),
            pl.BlockSpec((1, HID), lambda i: (0, 0)),
            pl.BlockSpec((HID, HID), lambda i: (0, 0)),
            pl.BlockSpec((1, HID), lambda i: (0, 0)),
        ],
        out_specs=[
            pl.BlockSpec((B, HID), lambda i: (i, 0)),
            pl.BlockSpec((8, HID), lambda i: (0, 0)),
        ],
        out_shape=[
            jax.ShapeDtypeStruct((NP, HID), jnp.float32),
            jax.ShapeDtypeStruct((8, HID), jnp.float32),
        ],
    )(x0, x1, a0, a1, w1p, b1, w2, b2)


def _bn_body(h2_ref, st_ref, g_ref, bt_ref, y0_ref, y1_ref, pool_ref, *, B):
    inv_n = 1.0 / N
    mean = st_ref[0:1, :] * inv_n
    var = st_ref[1:2, :] * inv_n - mean * mean
    inv = jax.lax.rsqrt(var + 1e-5)
    y = jnp.maximum((h2_ref[...] - mean) * inv * g_ref[...] + bt_ref[...], 0.0)
    y0_ref[...] = y[:, :128]
    y1_ref[...] = y[:, 128:]
    gid = pl.program_id(0)
    rows = gid * B + jax.lax.broadcasted_iota(jnp.int32, (B, 1), 0)
    ym = y * (rows < N).astype(jnp.float32)

    @pl.when(gid == 0)
    def _():
        pool_ref[...] = jnp.zeros_like(pool_ref)

    pool_ref[0:1, :] += jnp.sum(ym, axis=0, keepdims=True)


def _bn_apply(h2, st, gamma, beta):
    B = 512
    return pl.pallas_call(
        functools.partial(_bn_body, B=B),
        grid=(NP // B,),
        in_specs=[
            pl.BlockSpec((B, HID), lambda i: (i, 0)),
            pl.BlockSpec((8, HID), lambda i: (0, 0)),
            pl.BlockSpec((1, HID), lambda i: (0, 0)),
            pl.BlockSpec((1, HID), lambda i: (0, 0)),
        ],
        out_specs=[
            pl.BlockSpec((B, 128), lambda i: (i, 0)),
            pl.BlockSpec((B, 128), lambda i: (i, 0)),
            pl.BlockSpec((8, HID), lambda i: (0, 0)),
        ],
        out_shape=[
            jax.ShapeDtypeStruct((NP, 128), jnp.float32),
            jax.ShapeDtypeStruct((NP, 128), jnp.float32),
            jax.ShapeDtypeStruct((8, HID), jnp.float32),
        ],
    )(h2, st, gamma, beta)


def _head_body(pool_ref, wm1_ref, bm1_ref, wm2_ref, bm2_ref, out_ref):
    g = pool_ref[...] * (1.0 / N)
    a = jnp.maximum(
        jnp.dot(g, wm1_ref[...], preferred_element_type=jnp.float32) + bm1_ref[...], 0.0)
    out_ref[...] = jnp.dot(a, wm2_ref[...], preferred_element_type=jnp.float32) + bm2_ref[...]


def _head(pool, wm1, bm1, wm2p, bm2p):
    return pl.pallas_call(
        _head_body,
        grid=(1,),
        in_specs=[
            pl.BlockSpec((8, HID), lambda i: (0, 0)),
            pl.BlockSpec((HID, HID), lambda i: (0, 0)),
            pl.BlockSpec((1, HID), lambda i: (0, 0)),
            pl.BlockSpec((HID, 128), lambda i: (0, 0)),
            pl.BlockSpec((1, 128), lambda i: (0, 0)),
        ],
        out_specs=pl.BlockSpec((8, 128), lambda i: (0, 0)),
        out_shape=jax.ShapeDtypeStruct((8, 128), jnp.float32),
    )(pool, wm1, bm1, wm2p, bm2p)


def kernel(atomic_number, other_feats, edge_index, edge_attr, params):
    layers = params["layers"]
    src = edge_index[0].astype(jnp.int32)
    dst = edge_index[1].astype(jnp.int32)

    an2d = jnp.pad(atomic_number.astype(jnp.int32), (0, NP - N)).reshape(NP, 1)
    of_pad = jnp.pad(other_feats, ((0, NP - N), (0, 0)))
    emb_pad = jnp.pad(params["emb"], ((0, NUM_AT_PAD - 100), (0, 0)))
    w0p = jnp.pad(layers[0]["We"], ((0, 0), (0, 128 - 72)))
    wcat = jnp.concatenate([w0p] + [layers[i]["We"] for i in (1, 2, 3)], axis=1)

    x0 = _embed(an2d, of_pad, emb_pad)
    x1 = None
    ea_pad = jnp.pad(edge_attr, ((0, EP - E), (0, 0)))
    e_parts = None

    zrows = jnp.zeros((_SC_RPT, 128), jnp.float32)
    srcp = jnp.pad(src, (0, EP - E))
    dstp = jnp.pad(dst, (0, EP - E), constant_values=N + 8)
    perm = jnp.argsort(dstp, stable=True)
    srcp = srcp[perm]
    dstp = dstp[perm]
    e_parts = _edge_mlp(ea_pad[perm], wcat)
    srcr = srcp.reshape(EP // 40, 40)
    dstr = dstp.reshape(EP // 40, 40)
    pool = None
    for li, l in enumerate(layers):
        if li == 0:
            a0, a1 = _sc_msg0()(x0, srcr, dstr, e_parts[0], zrows)
        else:
            a0, a1 = _sc_msg()(x0, x1, srcr, dstr,
                               e_parts[2 * li - 1], e_parts[2 * li], zrows)

        din = 72 if li == 0 else HID
        din_pad = 128 if li == 0 else HID
        w1p = jnp.pad(l["W1"], ((0, din_pad - din), (0, 0)))
        h2, st = _node_mlp(x0, x0 if x1 is None else x1, a0, a1, w1p,
                           l["b1"].reshape(1, HID), l["W2"],
                           l["b2"].reshape(1, HID), li == 0)
        x0, x1, pool = _bn_apply(h2, st, l["gamma"].reshape(1, HID),
                                 l["beta"].reshape(1, HID))

    wm2p = jnp.pad(params["Wm2"], ((0, 0), (0, 127)))
    bm2p = jnp.pad(params["bm2"].reshape(1, 1), ((0, 0), (0, 127)))
    out = _head(pool, params["Wm1"], params["bm1"].reshape(1, HID), wm2p, bm2p)
    return out[0, 0].reshape(1)
